# baseline (speedup 1.0000x reference)
import functools

import jax
import jax.numpy as jnp
from jax.experimental import pallas as pl
from jax.experimental.pallas import tpu as pltpu
from jax.experimental.pallas import tpu_sc as plsc

DEPTH = 3
N_MOLS = 100

_NC = 2
_NS = 16
_NW = _NC * _NS
_GCHUNK = 80
_NBUF = 5


def _as_u32(x):
    n, d = x.shape
    return jax.lax.bitcast_convert_type(x.reshape(n, d // 2, 2), jnp.int32)


def _from_u32(x):
    n, d2 = x.shape
    return jax.lax.bitcast_convert_type(x, jnp.bfloat16).reshape(n, 2 * d2)


def _sc_gather(table, idx):
    n = idx.shape[0]
    d = table.shape[1]
    per_w = n // _NW
    group = _GCHUNK * _NBUF
    assert per_w * _NW == n and per_w % group == 0 and _GCHUNK % 8 == 0
    n_groups = per_w // group
    mesh = plsc.VectorSubcoreMesh(core_axis_name="c", subcore_axis_name="s")

    @functools.partial(
        pl.kernel,
        mesh=mesh,
        out_type=jax.ShapeDtypeStruct((n, d), table.dtype),
        scratch_types=[
            pltpu.VMEM((group,), jnp.int32),
            pltpu.VMEM((_NBUF, _GCHUNK, d), table.dtype),
            pltpu.SemaphoreType.DMA,
        ]
        + [pltpu.SemaphoreType.DMA] * _NBUF,
    )
    def k(table_hbm, idx_hbm, out_hbm, idx_v, rows_v, gsem, *wsems):
        wid = jax.lax.axis_index("s") * _NC + jax.lax.axis_index("c")
        base = wid * per_w

        def do_group(g, wait_wb):
            gbase = pl.multiple_of(base + g * group, 8)
            if wait_wb:
                for b in range(_NBUF):
                    off = pl.multiple_of(gbase + b * _GCHUNK, 8)
                    pltpu.make_async_copy(
                        rows_v.at[b], out_hbm.at[pl.ds(off, _GCHUNK)], wsems[b]
                    ).wait()
            pltpu.sync_copy(idx_hbm.at[pl.ds(gbase, group)], idx_v)
            handles = [
                pltpu.async_copy(
                    table_hbm.at[idx_v.at[pl.ds(b * _GCHUNK, _GCHUNK)]],
                    rows_v.at[b],
                    gsem,
                )
                for b in range(_NBUF)
            ]
            for b, h in enumerate(handles):
                h.wait()
                off = pl.multiple_of(gbase + b * _GCHUNK, 8)
                pltpu.async_copy(
                    rows_v.at[b], out_hbm.at[pl.ds(off, _GCHUNK)], wsems[b]
                )

        do_group(0, False)

        @pl.loop(1, n_groups)
        def _(g):
            do_group(g, True)

        last = pl.multiple_of(base + (n_groups - 1) * group, 8)
        for b in range(_NBUF):
            off = pl.multiple_of(last + b * _GCHUNK, 8)
            pltpu.make_async_copy(
                rows_v.at[b], out_hbm.at[pl.ds(off, _GCHUNK)], wsems[b]
            ).wait()

    return k(table, idx)


def _gather_bf16(table_bf16, idx):
    return _from_u32(_sc_gather(_as_u32(table_bf16), idx))


_BOND_BLK = 2000
_ATOM_BLK = 200
_BF = jnp.bfloat16


def _tc_init(f_bonds, w_i_t, w_h_t):
    n, fdim = f_bonds.shape
    h = w_i_t.shape[1]

    def body(fb, wi, wh, inp_ref, msgh_ref):
        inp = jnp.dot(
            fb[...].astype(_BF), wi[...], preferred_element_type=jnp.float32
        )
        inp_ref[...] = inp.astype(_BF)
        msgh_ref[...] = jnp.dot(
            jnp.maximum(inp, 0.0).astype(_BF), wh[...],
            preferred_element_type=jnp.float32,
        ).astype(_BF)

    return pl.pallas_call(
        body,
        grid=(n // _BOND_BLK,),
        in_specs=[
            pl.BlockSpec((_BOND_BLK, fdim), lambda i: (i, 0)),
            pl.BlockSpec((fdim, h), lambda i: (0, 0)),
            pl.BlockSpec((h, h), lambda i: (0, 0)),
        ],
        out_specs=[
            pl.BlockSpec((_BOND_BLK, h), lambda i: (i, 0)),
            pl.BlockSpec((_BOND_BLK, h), lambda i: (i, 0)),
        ],
        out_shape=[
            jax.ShapeDtypeStruct((n, h), _BF),
            jax.ShapeDtypeStruct((n, h), _BF),
        ],
    )(f_bonds, w_i_t, w_h_t)


def _tc_segsum(nei):
    a, k, h = nei.shape

    def body(n_ref, o_ref):
        o_ref[...] = jnp.sum(
            n_ref[...].astype(jnp.float32), axis=1
        ).astype(_BF)

    return pl.pallas_call(
        body,
        grid=(a // _ATOM_BLK,),
        in_specs=[pl.BlockSpec((_ATOM_BLK, k, h), lambda i: (i, 0, 0))],
        out_specs=pl.BlockSpec((_ATOM_BLK, h), lambda i: (i, 0)),
        out_shape=jax.ShapeDtypeStruct((a, h), _BF),
    )(nei)


def _tc_combine(inp, g1, rev, w_h_t):
    n, h = inp.shape
    matmul = w_h_t is not None

    def body(*refs):
        if matmul:
            inp_ref, g1_ref, rev_ref, wh_ref, o_ref = refs
        else:
            inp_ref, g1_ref, rev_ref, o_ref = refs
        m = jnp.maximum(
            inp_ref[...].astype(jnp.float32)
            + g1_ref[...].astype(jnp.float32)
            - rev_ref[...].astype(jnp.float32),
            0.0,
        )
        if matmul:
            m = jnp.dot(
                m.astype(_BF), wh_ref[...], preferred_element_type=jnp.float32
            )
        o_ref[...] = m.astype(_BF)

    row_spec = pl.BlockSpec((_BOND_BLK, h), lambda i: (i, 0))
    in_specs = [row_spec, row_spec, row_spec]
    args = [inp, g1, rev]
    if matmul:
        in_specs.append(pl.BlockSpec((h, h), lambda i: (0, 0)))
        args.append(w_h_t)
    return pl.pallas_call(
        body,
        grid=(n // _BOND_BLK,),
        in_specs=in_specs,
        out_specs=row_spec,
        out_shape=jax.ShapeDtypeStruct((n, h), _BF),
    )(*args)


def _tc_readout(f_atoms, nei, wo1_t, wo2_t, bias):
    a, fdim = f_atoms.shape
    _, k, h = nei.shape

    def body(fa_ref, n_ref, w1_ref, w2_ref, b_ref, o_ref):
        am = jnp.sum(n_ref[...].astype(jnp.float32), axis=1) * (1.0 / k)
        acc = jnp.dot(
            fa_ref[...].astype(_BF), w1_ref[...],
            preferred_element_type=jnp.float32,
        )
        acc += jnp.dot(
            am.astype(_BF), w2_ref[...], preferred_element_type=jnp.float32
        )
        o_ref[...] = jnp.maximum(acc + b_ref[...], 0.0)

    return pl.pallas_call(
        body,
        grid=(a // _ATOM_BLK,),
        in_specs=[
            pl.BlockSpec((_ATOM_BLK, fdim), lambda i: (i, 0)),
            pl.BlockSpec((_ATOM_BLK, k, h), lambda i: (i, 0, 0)),
            pl.BlockSpec((fdim, h), lambda i: (0, 0)),
            pl.BlockSpec((h, h), lambda i: (0, 0)),
            pl.BlockSpec((1, h), lambda i: (0, 0)),
        ],
        out_specs=pl.BlockSpec((_ATOM_BLK, h), lambda i: (i, 0)),
        out_shape=jax.ShapeDtypeStruct((a, h), jnp.float32),
    )(f_atoms, nei, wo1_t, wo2_t, bias)


def kernel(f_atoms, f_bonds, a2b, b2a, b2revb, W_i, W_h, W_o_w, W_o_b):
    n_atoms, atom_fdim = f_atoms.shape
    max_nb = a2b.shape[1]
    h = W_i.shape[0]

    a2b_flat = a2b.reshape(-1).astype(jnp.int32)
    b2a = b2a.astype(jnp.int32)
    b2revb = b2revb.astype(jnp.int32)
    w_i_t = W_i.T.astype(_BF)
    w_h_t = W_h.T.astype(_BF)
    wo1_t = W_o_w[:, :atom_fdim].T.astype(_BF)
    wo2_t = W_o_w[:, atom_fdim:].T.astype(_BF)
    bias = W_o_b.reshape(1, h)

    inp, msgh = _tc_init(f_bonds, w_i_t, w_h_t)
    message = None
    for t in range(DEPTH - 1):
        nei = _gather_bf16(msgh, a2b_flat)
        rev = _gather_bf16(msgh, b2revb)
        amh = _tc_segsum(nei.reshape(n_atoms, max_nb, h))
        g1 = _gather_bf16(amh, b2a)
        if t == DEPTH - 2:
            message = _tc_combine(inp, g1, rev, None)
        else:
            msgh = _tc_combine(inp, g1, rev, w_h_t)

    nei = _gather_bf16(message, a2b_flat)
    out = _tc_readout(
        f_atoms, nei.reshape(n_atoms, max_nb, h), wo1_t, wo2_t, bias
    )
    return out.reshape(N_MOLS, n_atoms // N_MOLS, h)


# speedup vs baseline: 8.8203x; 8.8203x over previous
import functools

import jax
import jax.numpy as jnp
from jax.experimental import pallas as pl
from jax.experimental.pallas import tpu as pltpu
from jax.experimental.pallas import tpu_sc as plsc

DEPTH = 3
N_MOLS = 100

_NC = 2
_NS = 16
_NW = _NC * _NS
_GCHUNK = 80
_NBUF = 5

_BF = jnp.bfloat16


def _rt16(x):
    u = jax.lax.bitcast_convert_type(x, jnp.uint32)
    u = u + jnp.uint32(0x7FFF) + ((u >> 16) & jnp.uint32(1))
    return u >> 16


def _pack2(lo, hi):
    return jax.lax.bitcast_convert_type(
        _rt16(lo) | (_rt16(hi) << 16), jnp.int32
    )


def _unpack(p):
    u = jax.lax.bitcast_convert_type(p, jnp.uint32)
    lo = jax.lax.bitcast_convert_type(u << 16, jnp.float32)
    hi = jax.lax.bitcast_convert_type(
        u & jnp.uint32(0xFFFF0000), jnp.float32
    )
    return lo, hi


def _sc_gather(table, idx):
    n = idx.shape[0]
    d = table.shape[1]
    per_w = n // _NW
    group = _GCHUNK * _NBUF
    assert per_w * _NW == n and per_w % group == 0 and _GCHUNK % 8 == 0
    n_groups = per_w // group
    mesh = plsc.VectorSubcoreMesh(core_axis_name="c", subcore_axis_name="s")

    @functools.partial(
        pl.kernel,
        mesh=mesh,
        out_type=jax.ShapeDtypeStruct((n, d), table.dtype),
        scratch_types=[
            pltpu.VMEM((group,), jnp.int32),
            pltpu.VMEM((_NBUF, _GCHUNK, d), table.dtype),
            pltpu.SemaphoreType.DMA,
        ]
        + [pltpu.SemaphoreType.DMA] * _NBUF,
    )
    def k(table_hbm, idx_hbm, out_hbm, idx_v, rows_v, gsem, *wsems):
        wid = jax.lax.axis_index("s") * _NC + jax.lax.axis_index("c")
        base = wid * per_w

        def do_group(g, wait_wb):
            gbase = pl.multiple_of(base + g * group, 8)
            if wait_wb:
                for b in range(_NBUF):
                    off = pl.multiple_of(gbase + b * _GCHUNK, 8)
                    pltpu.make_async_copy(
                        rows_v.at[b], out_hbm.at[pl.ds(off, _GCHUNK)], wsems[b]
                    ).wait()
            pltpu.sync_copy(idx_hbm.at[pl.ds(gbase, group)], idx_v)
            handles = [
                pltpu.async_copy(
                    table_hbm.at[idx_v.at[pl.ds(b * _GCHUNK, _GCHUNK)]],
                    rows_v.at[b],
                    gsem,
                )
                for b in range(_NBUF)
            ]
            for b, h in enumerate(handles):
                h.wait()
                off = pl.multiple_of(gbase + b * _GCHUNK, 8)
                pltpu.async_copy(
                    rows_v.at[b], out_hbm.at[pl.ds(off, _GCHUNK)], wsems[b]
                )

        do_group(0, False)

        @pl.loop(1, n_groups)
        def _(g):
            do_group(g, True)

        last = pl.multiple_of(base + (n_groups - 1) * group, 8)
        for b in range(_NBUF):
            off = pl.multiple_of(last + b * _GCHUNK, 8)
            pltpu.make_async_copy(
                rows_v.at[b], out_hbm.at[pl.ds(off, _GCHUNK)], wsems[b]
            ).wait()

    return k(table, idx)


_BOND_BLK = 2000
_ATOM_BLK = 200


def _tc_init(f_bonds, w_i_t, w_h_t):
    n, fdim = f_bonds.shape
    h = w_i_t.shape[1]
    hh = h // 2

    def body(fb, wi, wh, inp_ref, msgh_ref):
        inp = jnp.dot(
            fb[...].astype(_BF), wi[...], preferred_element_type=jnp.float32
        )
        inp_ref[...] = _pack2(inp[:, :hh], inp[:, hh:])
        msgh = jnp.dot(
            jnp.maximum(inp, 0.0).astype(_BF), wh[...],
            preferred_element_type=jnp.float32,
        )
        msgh_ref[...] = _pack2(msgh[:, :hh], msgh[:, hh:])

    return pl.pallas_call(
        body,
        grid=(n // _BOND_BLK,),
        in_specs=[
            pl.BlockSpec((_BOND_BLK, fdim), lambda i: (i, 0)),
            pl.BlockSpec((fdim, h), lambda i: (0, 0)),
            pl.BlockSpec((h, h), lambda i: (0, 0)),
        ],
        out_specs=[
            pl.BlockSpec((_BOND_BLK, hh), lambda i: (i, 0)),
            pl.BlockSpec((_BOND_BLK, hh), lambda i: (i, 0)),
        ],
        out_shape=[
            jax.ShapeDtypeStruct((n, hh), jnp.int32),
            jax.ShapeDtypeStruct((n, hh), jnp.int32),
        ],
    )(f_bonds, w_i_t, w_h_t)


def _tc_segsum(nei):
    a, k, hh = nei.shape

    def body(n_ref, o_ref):
        lo, hi = _unpack(n_ref[...])
        o_ref[...] = _pack2(jnp.sum(lo, axis=1), jnp.sum(hi, axis=1))

    return pl.pallas_call(
        body,
        grid=(a // _ATOM_BLK,),
        in_specs=[pl.BlockSpec((_ATOM_BLK, k, hh), lambda i: (i, 0, 0))],
        out_specs=pl.BlockSpec((_ATOM_BLK, hh), lambda i: (i, 0)),
        out_shape=jax.ShapeDtypeStruct((a, hh), jnp.int32),
    )(nei)


def _tc_combine(inp, g1, rev, w_h_t):
    n, hh = inp.shape
    matmul = w_h_t is not None

    def body(*refs):
        if matmul:
            inp_ref, g1_ref, rev_ref, wh_ref, o_ref = refs
        else:
            inp_ref, g1_ref, rev_ref, o_ref = refs
        i_lo, i_hi = _unpack(inp_ref[...])
        g_lo, g_hi = _unpack(g1_ref[...])
        r_lo, r_hi = _unpack(rev_ref[...])
        m_lo = jnp.maximum(i_lo + g_lo - r_lo, 0.0)
        m_hi = jnp.maximum(i_hi + g_hi - r_hi, 0.0)
        if matmul:
            wh = wh_ref[...]
            out = jnp.dot(
                m_lo.astype(_BF), wh[:hh], preferred_element_type=jnp.float32
            ) + jnp.dot(
                m_hi.astype(_BF), wh[hh:], preferred_element_type=jnp.float32
            )
            o_ref[...] = _pack2(out[:, :hh], out[:, hh:])
        else:
            o_ref[...] = _pack2(m_lo, m_hi)

    row_spec = pl.BlockSpec((_BOND_BLK, hh), lambda i: (i, 0))
    in_specs = [row_spec, row_spec, row_spec]
    args = [inp, g1, rev]
    if matmul:
        in_specs.append(pl.BlockSpec((2 * hh, 2 * hh), lambda i: (0, 0)))
        args.append(w_h_t)
    return pl.pallas_call(
        body,
        grid=(n // _BOND_BLK,),
        in_specs=in_specs,
        out_specs=row_spec,
        out_shape=jax.ShapeDtypeStruct((n, hh), jnp.int32),
    )(*args)


def _tc_readout(f_atoms, nei, wo1_t, wo2_t, bias):
    a, fdim = f_atoms.shape
    _, k, hh = nei.shape
    h = 2 * hh

    def body(fa_ref, n_ref, w1_ref, w2_ref, b_ref, o_ref):
        lo, hi = _unpack(n_ref[...])
        am_lo = jnp.sum(lo, axis=1) * (1.0 / k)
        am_hi = jnp.sum(hi, axis=1) * (1.0 / k)
        w2 = w2_ref[...]
        acc = jnp.dot(
            fa_ref[...].astype(_BF), w1_ref[...],
            preferred_element_type=jnp.float32,
        )
        acc += jnp.dot(
            am_lo.astype(_BF), w2[:hh], preferred_element_type=jnp.float32
        )
        acc += jnp.dot(
            am_hi.astype(_BF), w2[hh:], preferred_element_type=jnp.float32
        )
        o_ref[...] = jnp.maximum(acc + b_ref[...], 0.0)

    return pl.pallas_call(
        body,
        grid=(a // _ATOM_BLK,),
        in_specs=[
            pl.BlockSpec((_ATOM_BLK, fdim), lambda i: (i, 0)),
            pl.BlockSpec((_ATOM_BLK, k, hh), lambda i: (i, 0, 0)),
            pl.BlockSpec((fdim, h), lambda i: (0, 0)),
            pl.BlockSpec((h, h), lambda i: (0, 0)),
            pl.BlockSpec((1, h), lambda i: (0, 0)),
        ],
        out_specs=pl.BlockSpec((_ATOM_BLK, h), lambda i: (i, 0)),
        out_shape=jax.ShapeDtypeStruct((a, h), jnp.float32),
    )(f_atoms, nei, wo1_t, wo2_t, bias)


def kernel(f_atoms, f_bonds, a2b, b2a, b2revb, W_i, W_h, W_o_w, W_o_b):
    n_atoms, atom_fdim = f_atoms.shape
    max_nb = a2b.shape[1]
    h = W_i.shape[0]

    a2b_flat = a2b.reshape(-1).astype(jnp.int32)
    b2a = b2a.astype(jnp.int32)
    b2revb = b2revb.astype(jnp.int32)
    w_i_t = W_i.T.astype(_BF)
    w_h_t = W_h.T.astype(_BF)
    wo1_t = W_o_w[:, :atom_fdim].T.astype(_BF)
    wo2_t = W_o_w[:, atom_fdim:].T.astype(_BF)
    bias = W_o_b.reshape(1, h)

    inp, msgh = _tc_init(f_bonds, w_i_t, w_h_t)
    message = None
    for t in range(DEPTH - 1):
        nei = _sc_gather(msgh, a2b_flat)
        rev = _sc_gather(msgh, b2revb)
        amh = _tc_segsum(nei.reshape(n_atoms, max_nb, h // 2))
        g1 = _sc_gather(amh, b2a)
        if t == DEPTH - 2:
            message = _tc_combine(inp, g1, rev, None)
        else:
            msgh = _tc_combine(inp, g1, rev, w_h_t)

    nei = _sc_gather(message, a2b_flat)
    out = _tc_readout(
        f_atoms, nei.reshape(n_atoms, max_nb, h // 2), wo1_t, wo2_t, bias
    )
    return out.reshape(N_MOLS, n_atoms // N_MOLS, h)
